# baseline (speedup 1.0000x reference)
import functools

import jax
import jax.numpy as jnp
from jax import lax
from jax.experimental import pallas as pl
from jax.experimental.pallas import tpu as pltpu
from jax.experimental.pallas import tpu_sc as plsc

LANES = 16


def _mm(a, b):
    return jnp.dot(a, b, preferred_element_type=jnp.float32)


def _enc_body(x_ref, w1_ref, b1_ref, w2_ref, b2_ref, o_ref):
    h = jnp.maximum(_mm(x_ref[...], w1_ref[...]) + b1_ref[...], 0.0)
    o_ref[...] = _mm(h, w2_ref[...]) + b2_ref[...]


def _encoder(x, w1, b1, w2, b2, bn):
    n, d = x.shape
    h = w2.shape[1]
    return pl.pallas_call(
        _enc_body,
        grid=(n // bn,),
        in_specs=[
            pl.BlockSpec((bn, d), lambda i: (i, 0)),
            pl.BlockSpec((d, h), lambda i: (0, 0)),
            pl.BlockSpec((1, h), lambda i: (0, 0)),
            pl.BlockSpec((h, h), lambda i: (0, 0)),
            pl.BlockSpec((1, h), lambda i: (0, 0)),
        ],
        out_specs=pl.BlockSpec((bn, h), lambda i: (i, 0)),
        out_shape=jax.ShapeDtypeStruct((n, h), jnp.float32),
    )(x, w1, b1.reshape(1, -1), w2, b2.reshape(1, -1))


def _pq_body(h_ref, wa_ref, wb_ref, p_ref, q_ref):
    hv = h_ref[...]
    p_ref[...] = _mm(hv, wa_ref[...])
    q_ref[...] = _mm(hv, wb_ref[...])


def _pq(h, wa, wb, bn):
    n, d = h.shape
    return pl.pallas_call(
        _pq_body,
        grid=(n // bn,),
        in_specs=[
            pl.BlockSpec((bn, d), lambda i: (i, 0)),
            pl.BlockSpec((d, d), lambda i: (0, 0)),
            pl.BlockSpec((d, d), lambda i: (0, 0)),
        ],
        out_specs=[
            pl.BlockSpec((bn, d), lambda i: (i, 0)),
            pl.BlockSpec((bn, d), lambda i: (i, 0)),
        ],
        out_shape=[
            jax.ShapeDtypeStruct((n, d), jnp.float32),
            jax.ShapeDtypeStruct((n, d), jnp.float32),
        ],
    )(h, wa, wb)


def _r_body(ea_ref, eew_ref, eeb_ref, wc_ref, mb1_ref, o_ref):
    ef = jnp.maximum(_mm(ea_ref[...], eew_ref[...]) + eeb_ref[...], 0.0)
    o_ref[...] = _mm(ef, wc_ref[...]) + mb1_ref[...]


def _r_term(edge_attr, eew, eeb, wc, mb1, be):
    e, de = edge_attr.shape
    h = wc.shape[1]
    return pl.pallas_call(
        _r_body,
        grid=(e // be,),
        in_specs=[
            pl.BlockSpec((be, de), lambda i: (i, 0)),
            pl.BlockSpec((de, h), lambda i: (0, 0)),
            pl.BlockSpec((1, h), lambda i: (0, 0)),
            pl.BlockSpec((h, h), lambda i: (0, 0)),
            pl.BlockSpec((1, h), lambda i: (0, 0)),
        ],
        out_specs=pl.BlockSpec((be, h), lambda i: (i, 0)),
        out_shape=jax.ShapeDtypeStruct((e, h), jnp.float32),
    )(edge_attr, eew, eeb.reshape(1, -1), wc, mb1.reshape(1, -1))


def _node_body(h_ref, s0_ref, s1_ref, d0_ref, d1_ref, mw2_ref, mb2_ref,
               uwh_ref, uwa_ref, ub_ref, g_ref, b_ref, o_ref):
    hv = h_ref[...]
    s = s0_ref[0] + s1_ref[0]
    deg = d0_ref[0][:, :1] + d1_ref[0][:, :1]
    cnt = jnp.maximum(deg, 1.0)
    ind = jnp.minimum(deg, 1.0)
    agg = _mm(s / cnt, mw2_ref[...]) + mb2_ref[...] * ind
    upd = jnp.maximum(_mm(hv, uwh_ref[...]) + _mm(agg, uwa_ref[...])
                      + ub_ref[...], 0.0)
    h2 = hv + upd
    mu = jnp.mean(h2, axis=-1, keepdims=True)
    var = jnp.mean((h2 - mu) ** 2, axis=-1, keepdims=True)
    o_ref[...] = (h2 - mu) * lax.rsqrt(var + 1e-5) * g_ref[...] + b_ref[...]


def _node_update(h, s2, d2, mw2, mb2, uwh, uwa, ub, g, b, bn):
    n, dd = h.shape
    dcols = d2.shape[-1]
    return pl.pallas_call(
        _node_body,
        grid=(n // bn,),
        in_specs=[
            pl.BlockSpec((bn, dd), lambda i: (i, 0)),
            pl.BlockSpec((1, bn, dd), lambda i: (0, i, 0)),
            pl.BlockSpec((1, bn, dd), lambda i: (1, i, 0)),
            pl.BlockSpec((1, bn, dcols), lambda i: (0, i, 0)),
            pl.BlockSpec((1, bn, dcols), lambda i: (1, i, 0)),
            pl.BlockSpec((dd, dd), lambda i: (0, 0)),
            pl.BlockSpec((1, dd), lambda i: (0, 0)),
            pl.BlockSpec((dd, dd), lambda i: (0, 0)),
            pl.BlockSpec((dd, dd), lambda i: (0, 0)),
            pl.BlockSpec((1, dd), lambda i: (0, 0)),
            pl.BlockSpec((1, dd), lambda i: (0, 0)),
            pl.BlockSpec((1, dd), lambda i: (0, 0)),
        ],
        out_specs=pl.BlockSpec((bn, dd), lambda i: (i, 0)),
        out_shape=jax.ShapeDtypeStruct((n, dd), jnp.float32),
    )(h, s2, s2, d2, d2, mw2, mb2.reshape(1, -1), uwh, uwa,
      ub.reshape(1, -1), g.reshape(1, -1), b.reshape(1, -1))


def _dec_body(h_ref, w1_ref, b1_ref, w2_ref, b2_ref, o_ref):
    p = jnp.maximum(_mm(h_ref[...], w1_ref[...]) + b1_ref[...], 0.0)
    o_ref[...] = _mm(p, w2_ref[...]) + b2_ref[...]


def _decoder(h, w1, b1, w2, b2, bn):
    n, d = h.shape
    return pl.pallas_call(
        _dec_body,
        grid=(n // bn,),
        in_specs=[
            pl.BlockSpec((bn, d), lambda i: (i, 0)),
            pl.BlockSpec((d, d), lambda i: (0, 0)),
            pl.BlockSpec((1, d), lambda i: (0, 0)),
            pl.BlockSpec((d, 1), lambda i: (0, 0)),
            pl.BlockSpec((1, 1), lambda i: (0, 0)),
        ],
        out_specs=pl.BlockSpec((bn, 1), lambda i: (i, 0)),
        out_shape=jax.ShapeDtypeStruct((n, 1), jnp.float32),
    )(h, w1, b1.reshape(1, -1), w2, b2.reshape(1, 1))


CH = 40
CHD = 40
NSET = 2
NSLOT = 4


def _sc_segment_relu_sum(p, q, r, dst, src, nc, ns):
    n, h = p.shape
    e = dst.shape[0]
    nw = nc * ns
    total_ch = e // CH
    base_ch, extra = divmod(total_ch, nw)
    groups = h // LANES
    zch = n // CH
    ztail = n - zch * CH
    mesh = plsc.VectorSubcoreMesh(core_axis_name="c", subcore_axis_name="s")

    @functools.partial(
        pl.kernel,
        out_type=jax.ShapeDtypeStruct((nc * n, h), jnp.float32),
        mesh=mesh,
        scratch_types=[
            [pltpu.VMEM((CH,), jnp.int32)] * NSLOT,
            [pltpu.VMEM((CH,), jnp.int32)] * NSLOT,
            [pltpu.VMEM((CH, h), jnp.float32)] * NSET,
            [pltpu.VMEM((CH, h), jnp.float32)] * NSET,
            [pltpu.VMEM((CH, h), jnp.float32)] * NSET,
            [pltpu.VMEM((CH, h), jnp.float32)] * NSET,
            pltpu.VMEM_SHARED((n, h), jnp.float32),
            [pltpu.SemaphoreType.DMA] * NSLOT,
            [pltpu.SemaphoreType.DMA] * NSLOT,
            [pltpu.SemaphoreType.DMA] * NSET,
            [pltpu.SemaphoreType.DMA] * NSET,
            [pltpu.SemaphoreType.DMA] * NSET,
            [pltpu.SemaphoreType.DMA] * NSET,
        ],
    )
    def k(p_hbm, q_hbm, r_hbm, dst_hbm, src_hbm, out_hbm,
          dsti, srci, pbuf, qbuf, rbuf, obuf, acc,
          semd, sems, semp, semq, semr, sema):
        cid = lax.axis_index("c")
        sid = lax.axis_index("s")
        wid = sid * nc + cid
        n_my = base_ch + jnp.where(wid < extra, 1, 0)

        def zrow(i, _):
            rr = i // groups
            jj = (i % groups) * LANES
            pbuf[0][rr, pl.ds(jj, LANES)] = jnp.zeros((LANES,), jnp.float32)
            return 0
        lax.fori_loop(0, CH * groups, zrow, 0)

        def zchunk(kk, _):
            m = sid + kk * ns
            @pl.when(m < zch)
            def _():
                pltpu.sync_copy(pbuf[0], acc.at[pl.ds(m * CH, CH)])
            if ztail:
                @pl.when(m == zch)
                def _():
                    pltpu.sync_copy(pbuf[0].at[pl.ds(0, ztail)],
                                    acc.at[pl.ds(zch * CH, ztail)])
            return 0
        lax.fori_loop(0, -(-(zch + (1 if ztail else 0)) // ns), zchunk, 0)
        plsc.subcore_barrier()

        def fire_idx(t, s):
            e0 = (wid + nw * t) * CH
            pltpu.async_copy(dst_hbm.at[pl.ds(e0, CH)], dsti[s], semd[s])
            pltpu.async_copy(src_hbm.at[pl.ds(e0, CH)], srci[s], sems[s])

        def wait_idx(s):
            pltpu.make_async_copy(dst_hbm.at[pl.ds(0, CH)], dsti[s],
                                  semd[s]).wait()
            pltpu.make_async_copy(src_hbm.at[pl.ds(0, CH)], srci[s],
                                  sems[s]).wait()

        def fire_gather(t, b, s):
            e0 = (wid + nw * t) * CH
            pltpu.async_copy(p_hbm.at[dsti[s]], pbuf[b], semp[b])
            pltpu.async_copy(q_hbm.at[srci[s]], qbuf[b], semq[b])
            pltpu.async_copy(r_hbm.at[pl.ds(e0, CH)], rbuf[b], semr[b])

        def wait_gather(b, s):
            pltpu.make_async_copy(p_hbm.at[dsti[s]], pbuf[b], semp[b]).wait()
            pltpu.make_async_copy(q_hbm.at[srci[s]], qbuf[b], semq[b]).wait()
            pltpu.make_async_copy(r_hbm.at[pl.ds(0, CH)], rbuf[b],
                                  semr[b]).wait()

        def fire_scatter(b, s):
            pltpu.async_copy(obuf[b], acc.at[dsti[s]], sema[b], add=True)

        def wait_scatter(b, s):
            pltpu.make_async_copy(obuf[b], acc.at[dsti[s]], sema[b]).wait()

        fire_idx(0, 0)
        fire_idx(1, 1)
        wait_idx(0)
        fire_gather(0, 0, 0)

        def chunk_body(t, b, s):
            bm1 = 1 - b
            sm1 = (s + NSLOT - 1) % NSLOT
            sp1 = (s + 1) % NSLOT
            sp2 = (s + 2) % NSLOT
            @pl.when(t >= 1)
            def _():
                wait_scatter(bm1, sm1)
            @pl.when(t + 1 < n_my)
            def _():
                wait_idx(sp1)
                fire_gather(t + 1, bm1, sp1)
            @pl.when(t + 2 < n_my)
            def _():
                fire_idx(t + 2, sp2)
            wait_gather(b, s)

            @plsc.parallel_loop(0, CH, unroll=4)
            def _(rr):
                for j in range(groups):
                    sl = pl.ds(j * LANES, LANES)
                    v = pbuf[b][rr, sl] + qbuf[b][rr, sl] + rbuf[b][rr, sl]
                    obuf[b][rr, sl] = jnp.maximum(v, 0.0)
            fire_scatter(b, s)

        def chunk(t, _):
            for kph in range(NSLOT):
                @pl.when(t % NSLOT == kph)
                def _(kph=kph):
                    chunk_body(t, kph % NSET, kph % NSLOT)
            return 0
        lax.fori_loop(0, n_my, chunk, 0)

        tl = n_my - 1
        for kph in range(NSLOT):
            @pl.when(tl % NSLOT == kph)
            def _(kph=kph):
                wait_scatter(kph % NSET, kph % NSLOT)

        plsc.subcore_barrier()

        def wchunk(kk, _):
            m = sid + kk * ns
            @pl.when(m < zch)
            def _():
                pltpu.sync_copy(acc.at[pl.ds(m * CH, CH)],
                                out_hbm.at[pl.ds(cid * n + m * CH, CH)])
            if ztail:
                @pl.when(m == zch)
                def _():
                    pltpu.sync_copy(acc.at[pl.ds(zch * CH, ztail)],
                                    out_hbm.at[pl.ds(cid * n + zch * CH,
                                                     ztail)])
            return 0
        lax.fori_loop(0, -(-(zch + (1 if ztail else 0)) // ns), wchunk, 0)

    return k(p, q, r, dst, src).reshape(nc, n, h)


def _sc_degree(dst, n, h, nc, ns):
    e = dst.shape[0]
    nw = nc * ns
    n_ch = (e // CHD) // nw
    n_rch = n // CHD
    rch_per_sub = -(-n_rch // ns)
    groups = h // LANES
    mesh = plsc.VectorSubcoreMesh(core_axis_name="c", subcore_axis_name="s")

    @functools.partial(
        pl.kernel,
        out_type=jax.ShapeDtypeStruct((nc * n, h), jnp.float32),
        mesh=mesh,
        scratch_types=[
            [pltpu.VMEM((CHD,), jnp.int32)] * 4,
            pltpu.VMEM((CHD, h), jnp.float32),
            pltpu.VMEM((CHD, h), jnp.float32),
            pltpu.VMEM_SHARED((n, h), jnp.float32),
            [pltpu.SemaphoreType.DMA] * 4,
            [pltpu.SemaphoreType.DMA] * 4,
        ],
    )
    def k(dst_hbm, out_hbm, dsti, ones_v, zeros_v, acc, semd, sema):
        cid = lax.axis_index("c")
        sid = lax.axis_index("s")
        wid = sid * nc + cid

        def fill(i, _):
            rr = i // groups
            jj = (i % groups) * LANES
            ones_v[rr, pl.ds(jj, LANES)] = jnp.ones((LANES,), jnp.float32)
            zeros_v[rr, pl.ds(jj, LANES)] = jnp.zeros((LANES,), jnp.float32)
            return 0
        lax.fori_loop(0, CHD * groups, fill, 0)

        def zchunk(kk, _):
            m = sid + kk * ns
            @pl.when(m < n_rch)
            def _():
                pltpu.sync_copy(zeros_v, acc.at[pl.ds(m * CHD, CHD)])
            return 0
        lax.fori_loop(0, rch_per_sub, zchunk, 0)
        plsc.subcore_barrier()

        def fire_idx(t, s):
            e0 = (wid + nw * t) * CHD
            pltpu.async_copy(dst_hbm.at[pl.ds(e0, CHD)], dsti[s], semd[s])

        def wait_idx(s):
            pltpu.make_async_copy(dst_hbm.at[pl.ds(0, CHD)], dsti[s],
                                  semd[s]).wait()

        def wait_scatter(s):
            pltpu.make_async_copy(ones_v, acc.at[dsti[s]], sema[s]).wait()

        fire_idx(0, 0)
        fire_idx(1, 1)

        def chunk_body(t, s):
            sp2 = (s + 2) % 4
            @pl.when(t >= 2)
            def _():
                wait_scatter(sp2)
            @pl.when(t + 2 < n_ch)
            def _():
                fire_idx(t + 2, sp2)
            wait_idx(s)
            pltpu.async_copy(ones_v, acc.at[dsti[s]], sema[s], add=True)

        def chunk(t, _):
            for kph in range(4):
                @pl.when(t % 4 == kph)
                def _(kph=kph):
                    chunk_body(t, kph)
            return 0
        lax.fori_loop(0, n_ch, chunk, 0)

        for tl in (n_ch - 1, n_ch - 2):
            for kph in range(4):
                @pl.when(tl % 4 == kph)
                def _(kph=kph):
                    wait_scatter(kph)

        plsc.subcore_barrier()

        def wchunk(kk, _):
            m = sid + kk * ns
            @pl.when(m < n_rch)
            def _():
                pltpu.sync_copy(acc.at[pl.ds(m * CHD, CHD)],
                                out_hbm.at[pl.ds(cid * n + m * CHD, CHD)])
            return 0
        lax.fori_loop(0, rch_per_sub, wchunk, 0)

    return k(dst).reshape(nc, n, h)


def kernel(x, edge_index, edge_attr, params):
    n, _ = x.shape
    e = edge_attr.shape[0]
    h = params["enc_w1"].shape[1]
    info = plsc.get_sparse_core_info()
    nc, ns = info.num_cores, info.num_subcores

    bn = 2000
    be = 4000

    src = edge_index[0]
    dst = edge_index[1]

    hv = _encoder(x, params["enc_w1"], params["enc_b1"],
                  params["enc_w2"], params["enc_b2"], bn)
    d2 = _sc_degree(dst, n, h, nc, ns)

    for lp in params["layers"]:
        wa = lp["mw1"][:h]
        wb = lp["mw1"][h:2 * h]
        wc = lp["mw1"][2 * h:]
        p, q = _pq(hv, wa, wb, bn)
        r = _r_term(edge_attr, params["ee_w"], params["ee_b"], wc,
                    lp["mb1"], be)
        s2 = _sc_segment_relu_sum(p, q, r, dst, src, nc, ns)
        hv = _node_update(hv, s2, d2, lp["mw2"], lp["mb2"],
                          lp["uw"][:h], lp["uw"][h:], lp["ub"],
                          lp["ln_g"], lp["ln_b"], bn)

    out = _decoder(hv, params["dec_w1"], params["dec_b1"],
                   params["dec_w2"], params["dec_b2"], bn)
    return out.reshape(n)


# speedup vs baseline: 1.0618x; 1.0618x over previous
import functools

import jax
import jax.numpy as jnp
from jax import lax
from jax.experimental import pallas as pl
from jax.experimental.pallas import tpu as pltpu
from jax.experimental.pallas import tpu_sc as plsc

LANES = 16


def _mm(a, b):
    return jnp.dot(a, b, preferred_element_type=jnp.float32)


def _enc_body(x_ref, w1_ref, b1_ref, w2_ref, b2_ref, o_ref):
    h = jnp.maximum(_mm(x_ref[...], w1_ref[...]) + b1_ref[...], 0.0)
    o_ref[...] = _mm(h, w2_ref[...]) + b2_ref[...]


def _encoder(x, w1, b1, w2, b2, bn):
    n, d = x.shape
    h = w2.shape[1]
    return pl.pallas_call(
        _enc_body,
        grid=(n // bn,),
        in_specs=[
            pl.BlockSpec((bn, d), lambda i: (i, 0)),
            pl.BlockSpec((d, h), lambda i: (0, 0)),
            pl.BlockSpec((1, h), lambda i: (0, 0)),
            pl.BlockSpec((h, h), lambda i: (0, 0)),
            pl.BlockSpec((1, h), lambda i: (0, 0)),
        ],
        out_specs=pl.BlockSpec((bn, h), lambda i: (i, 0)),
        out_shape=jax.ShapeDtypeStruct((n, h), jnp.float32),
    )(x, w1, b1.reshape(1, -1), w2, b2.reshape(1, -1))


def _pq_body(h_ref, wa_ref, wb_ref, p_ref, q_ref):
    hv = h_ref[...]
    p_ref[...] = _mm(hv, wa_ref[...])
    q_ref[...] = _mm(hv, wb_ref[...])


def _pq(h, wa, wb, bn):
    n, d = h.shape
    return pl.pallas_call(
        _pq_body,
        grid=(n // bn,),
        in_specs=[
            pl.BlockSpec((bn, d), lambda i: (i, 0)),
            pl.BlockSpec((d, d), lambda i: (0, 0)),
            pl.BlockSpec((d, d), lambda i: (0, 0)),
        ],
        out_specs=[
            pl.BlockSpec((bn, d), lambda i: (i, 0)),
            pl.BlockSpec((bn, d), lambda i: (i, 0)),
        ],
        out_shape=[
            jax.ShapeDtypeStruct((n, d), jnp.float32),
            jax.ShapeDtypeStruct((n, d), jnp.float32),
        ],
    )(h, wa, wb)


def _r_body(ea_ref, eew_ref, eeb_ref, wc_ref, mb1_ref, o_ref):
    ef = jnp.maximum(_mm(ea_ref[...], eew_ref[...]) + eeb_ref[...], 0.0)
    o_ref[...] = _mm(ef, wc_ref[...]) + mb1_ref[...]


def _r_term(edge_attr, eew, eeb, wc, mb1, be):
    e, de = edge_attr.shape
    h = wc.shape[1]
    return pl.pallas_call(
        _r_body,
        grid=(e // be,),
        in_specs=[
            pl.BlockSpec((be, de), lambda i: (i, 0)),
            pl.BlockSpec((de, h), lambda i: (0, 0)),
            pl.BlockSpec((1, h), lambda i: (0, 0)),
            pl.BlockSpec((h, h), lambda i: (0, 0)),
            pl.BlockSpec((1, h), lambda i: (0, 0)),
        ],
        out_specs=pl.BlockSpec((be, h), lambda i: (i, 0)),
        out_shape=jax.ShapeDtypeStruct((e, h), jnp.float32),
    )(edge_attr, eew, eeb.reshape(1, -1), wc, mb1.reshape(1, -1))


def _node_body(h_ref, s0_ref, s1_ref, d0_ref, d1_ref, mw2_ref, mb2_ref,
               uwh_ref, uwa_ref, ub_ref, g_ref, b_ref, o_ref):
    hv = h_ref[...]
    s = s0_ref[0] + s1_ref[0]
    deg = d0_ref[0][:, :1] + d1_ref[0][:, :1]
    cnt = jnp.maximum(deg, 1.0)
    ind = jnp.minimum(deg, 1.0)
    agg = _mm(s / cnt, mw2_ref[...]) + mb2_ref[...] * ind
    upd = jnp.maximum(_mm(hv, uwh_ref[...]) + _mm(agg, uwa_ref[...])
                      + ub_ref[...], 0.0)
    h2 = hv + upd
    mu = jnp.mean(h2, axis=-1, keepdims=True)
    var = jnp.mean((h2 - mu) ** 2, axis=-1, keepdims=True)
    o_ref[...] = (h2 - mu) * lax.rsqrt(var + 1e-5) * g_ref[...] + b_ref[...]


def _node_update(h, s2, d2, mw2, mb2, uwh, uwa, ub, g, b, bn):
    n, dd = h.shape
    dcols = d2.shape[-1]
    return pl.pallas_call(
        _node_body,
        grid=(n // bn,),
        in_specs=[
            pl.BlockSpec((bn, dd), lambda i: (i, 0)),
            pl.BlockSpec((1, bn, dd), lambda i: (0, i, 0)),
            pl.BlockSpec((1, bn, dd), lambda i: (1, i, 0)),
            pl.BlockSpec((1, bn, dcols), lambda i: (0, i, 0)),
            pl.BlockSpec((1, bn, dcols), lambda i: (1, i, 0)),
            pl.BlockSpec((dd, dd), lambda i: (0, 0)),
            pl.BlockSpec((1, dd), lambda i: (0, 0)),
            pl.BlockSpec((dd, dd), lambda i: (0, 0)),
            pl.BlockSpec((dd, dd), lambda i: (0, 0)),
            pl.BlockSpec((1, dd), lambda i: (0, 0)),
            pl.BlockSpec((1, dd), lambda i: (0, 0)),
            pl.BlockSpec((1, dd), lambda i: (0, 0)),
        ],
        out_specs=pl.BlockSpec((bn, dd), lambda i: (i, 0)),
        out_shape=jax.ShapeDtypeStruct((n, dd), jnp.float32),
    )(h, s2, s2, d2, d2, mw2, mb2.reshape(1, -1), uwh, uwa,
      ub.reshape(1, -1), g.reshape(1, -1), b.reshape(1, -1))


def _dec_body(h_ref, w1_ref, b1_ref, w2_ref, b2_ref, o_ref):
    p = jnp.maximum(_mm(h_ref[...], w1_ref[...]) + b1_ref[...], 0.0)
    o_ref[...] = _mm(p, w2_ref[...]) + b2_ref[...]


def _decoder(h, w1, b1, w2, b2, bn):
    n, d = h.shape
    return pl.pallas_call(
        _dec_body,
        grid=(n // bn,),
        in_specs=[
            pl.BlockSpec((bn, d), lambda i: (i, 0)),
            pl.BlockSpec((d, d), lambda i: (0, 0)),
            pl.BlockSpec((1, d), lambda i: (0, 0)),
            pl.BlockSpec((d, 1), lambda i: (0, 0)),
            pl.BlockSpec((1, 1), lambda i: (0, 0)),
        ],
        out_specs=pl.BlockSpec((bn, 1), lambda i: (i, 0)),
        out_shape=jax.ShapeDtypeStruct((n, 1), jnp.float32),
    )(h, w1, b1.reshape(1, -1), w2, b2.reshape(1, 1))


CH = 64
CHD = 40
NSET = 2
NSLOT = 4


def _sc_segment_relu_sum(p, q, r, dst, src, nc, ns):
    n, h = p.shape
    e = dst.shape[0]
    nw = nc * ns
    total_ch = e // CH
    base_ch, extra = divmod(total_ch, nw)
    groups = h // LANES
    zch = n // CH
    ztail = n - zch * CH
    mesh = plsc.VectorSubcoreMesh(core_axis_name="c", subcore_axis_name="s")

    @functools.partial(
        pl.kernel,
        out_type=jax.ShapeDtypeStruct((nc * n, h), jnp.float32),
        mesh=mesh,
        scratch_types=[
            [pltpu.VMEM((CH,), jnp.int32)] * NSLOT,
            [pltpu.VMEM((CH,), jnp.int32)] * NSLOT,
            [pltpu.VMEM((CH, h), jnp.float32)] * NSET,
            [pltpu.VMEM((CH, h), jnp.float32)] * NSET,
            [pltpu.VMEM((CH, h), jnp.float32)] * NSET,
            pltpu.VMEM_SHARED((n, h), jnp.float32),
            [pltpu.SemaphoreType.DMA] * NSLOT,
            [pltpu.SemaphoreType.DMA] * NSLOT,
            [pltpu.SemaphoreType.DMA] * NSET,
            [pltpu.SemaphoreType.DMA] * NSET,
            [pltpu.SemaphoreType.DMA] * NSET,
            [pltpu.SemaphoreType.DMA] * NSET,
        ],
    )
    def k(p_hbm, q_hbm, r_hbm, dst_hbm, src_hbm, out_hbm,
          dsti, srci, pbuf, qbuf, rbuf, acc,
          semd, sems, semp, semq, semr, sema):
        cid = lax.axis_index("c")
        sid = lax.axis_index("s")
        wid = sid * nc + cid
        n_my = base_ch + jnp.where(wid < extra, 1, 0)

        def zrow(i, _):
            rr = i // groups
            jj = (i % groups) * LANES
            pbuf[0][rr, pl.ds(jj, LANES)] = jnp.zeros((LANES,), jnp.float32)
            return 0
        lax.fori_loop(0, CH * groups, zrow, 0)

        def zchunk(kk, _):
            m = sid + kk * ns
            @pl.when(m < zch)
            def _():
                pltpu.sync_copy(pbuf[0], acc.at[pl.ds(m * CH, CH)])
            if ztail:
                @pl.when(m == zch)
                def _():
                    pltpu.sync_copy(pbuf[0].at[pl.ds(0, ztail)],
                                    acc.at[pl.ds(zch * CH, ztail)])
            return 0
        lax.fori_loop(0, -(-(zch + (1 if ztail else 0)) // ns), zchunk, 0)
        plsc.subcore_barrier()

        def fire_idx(t, s):
            e0 = (wid + nw * t) * CH
            pltpu.async_copy(dst_hbm.at[pl.ds(e0, CH)], dsti[s], semd[s])
            pltpu.async_copy(src_hbm.at[pl.ds(e0, CH)], srci[s], sems[s])

        def wait_idx(s):
            pltpu.make_async_copy(dst_hbm.at[pl.ds(0, CH)], dsti[s],
                                  semd[s]).wait()
            pltpu.make_async_copy(src_hbm.at[pl.ds(0, CH)], srci[s],
                                  sems[s]).wait()

        def fire_gather(t, b, s):
            e0 = (wid + nw * t) * CH
            pltpu.async_copy(p_hbm.at[dsti[s]], pbuf[b], semp[b])
            pltpu.async_copy(q_hbm.at[srci[s]], qbuf[b], semq[b])
            pltpu.async_copy(r_hbm.at[pl.ds(e0, CH)], rbuf[b], semr[b])

        def wait_gather(b, s):
            pltpu.make_async_copy(p_hbm.at[dsti[s]], pbuf[b], semp[b]).wait()
            pltpu.make_async_copy(q_hbm.at[srci[s]], qbuf[b], semq[b]).wait()
            pltpu.make_async_copy(r_hbm.at[pl.ds(0, CH)], rbuf[b],
                                  semr[b]).wait()

        def fire_scatter(b, s):
            pltpu.async_copy(pbuf[b], acc.at[dsti[s]], sema[b], add=True)

        def wait_scatter(b, s):
            pltpu.make_async_copy(pbuf[b], acc.at[dsti[s]], sema[b]).wait()

        fire_idx(0, 0)
        fire_idx(1, 1)
        wait_idx(0)
        fire_gather(0, 0, 0)

        def chunk_body(t, b, s):
            bm1 = 1 - b
            sm1 = (s + NSLOT - 1) % NSLOT
            sp1 = (s + 1) % NSLOT
            sp2 = (s + 2) % NSLOT
            @pl.when(t >= 1)
            def _():
                wait_scatter(bm1, sm1)
            @pl.when(t + 1 < n_my)
            def _():
                wait_idx(sp1)
                fire_gather(t + 1, bm1, sp1)
            @pl.when(t + 2 < n_my)
            def _():
                fire_idx(t + 2, sp2)
            wait_gather(b, s)

            @plsc.parallel_loop(0, CH, unroll=4)
            def _(rr):
                for j in range(groups):
                    sl = pl.ds(j * LANES, LANES)
                    v = pbuf[b][rr, sl] + qbuf[b][rr, sl] + rbuf[b][rr, sl]
                    pbuf[b][rr, sl] = jnp.maximum(v, 0.0)
            fire_scatter(b, s)

        def chunk(t, _):
            for kph in range(NSLOT):
                @pl.when(t % NSLOT == kph)
                def _(kph=kph):
                    chunk_body(t, kph % NSET, kph % NSLOT)
            return 0
        lax.fori_loop(0, n_my, chunk, 0)

        tl = n_my - 1
        for kph in range(NSLOT):
            @pl.when(tl % NSLOT == kph)
            def _(kph=kph):
                wait_scatter(kph % NSET, kph % NSLOT)

        plsc.subcore_barrier()

        def wchunk(kk, _):
            m = sid + kk * ns
            @pl.when(m < zch)
            def _():
                pltpu.sync_copy(acc.at[pl.ds(m * CH, CH)],
                                out_hbm.at[pl.ds(cid * n + m * CH, CH)])
            if ztail:
                @pl.when(m == zch)
                def _():
                    pltpu.sync_copy(acc.at[pl.ds(zch * CH, ztail)],
                                    out_hbm.at[pl.ds(cid * n + zch * CH,
                                                     ztail)])
            return 0
        lax.fori_loop(0, -(-(zch + (1 if ztail else 0)) // ns), wchunk, 0)

    return k(p, q, r, dst, src).reshape(nc, n, h)


def _sc_degree(dst, n, h, nc, ns):
    e = dst.shape[0]
    nw = nc * ns
    n_ch = (e // CHD) // nw
    n_rch = n // CHD
    rch_per_sub = -(-n_rch // ns)
    groups = h // LANES
    mesh = plsc.VectorSubcoreMesh(core_axis_name="c", subcore_axis_name="s")

    @functools.partial(
        pl.kernel,
        out_type=jax.ShapeDtypeStruct((nc * n, h), jnp.float32),
        mesh=mesh,
        scratch_types=[
            [pltpu.VMEM((CHD,), jnp.int32)] * 4,
            pltpu.VMEM((CHD, h), jnp.float32),
            pltpu.VMEM((CHD, h), jnp.float32),
            pltpu.VMEM_SHARED((n, h), jnp.float32),
            [pltpu.SemaphoreType.DMA] * 4,
            [pltpu.SemaphoreType.DMA] * 4,
        ],
    )
    def k(dst_hbm, out_hbm, dsti, ones_v, zeros_v, acc, semd, sema):
        cid = lax.axis_index("c")
        sid = lax.axis_index("s")
        wid = sid * nc + cid

        def fill(i, _):
            rr = i // groups
            jj = (i % groups) * LANES
            ones_v[rr, pl.ds(jj, LANES)] = jnp.ones((LANES,), jnp.float32)
            zeros_v[rr, pl.ds(jj, LANES)] = jnp.zeros((LANES,), jnp.float32)
            return 0
        lax.fori_loop(0, CHD * groups, fill, 0)

        def zchunk(kk, _):
            m = sid + kk * ns
            @pl.when(m < n_rch)
            def _():
                pltpu.sync_copy(zeros_v, acc.at[pl.ds(m * CHD, CHD)])
            return 0
        lax.fori_loop(0, rch_per_sub, zchunk, 0)
        plsc.subcore_barrier()

        def fire_idx(t, s):
            e0 = (wid + nw * t) * CHD
            pltpu.async_copy(dst_hbm.at[pl.ds(e0, CHD)], dsti[s], semd[s])

        def wait_idx(s):
            pltpu.make_async_copy(dst_hbm.at[pl.ds(0, CHD)], dsti[s],
                                  semd[s]).wait()

        def wait_scatter(s):
            pltpu.make_async_copy(ones_v, acc.at[dsti[s]], sema[s]).wait()

        fire_idx(0, 0)
        fire_idx(1, 1)

        def chunk_body(t, s):
            sp2 = (s + 2) % 4
            @pl.when(t >= 2)
            def _():
                wait_scatter(sp2)
            @pl.when(t + 2 < n_ch)
            def _():
                fire_idx(t + 2, sp2)
            wait_idx(s)
            pltpu.async_copy(ones_v, acc.at[dsti[s]], sema[s], add=True)

        def chunk(t, _):
            for kph in range(4):
                @pl.when(t % 4 == kph)
                def _(kph=kph):
                    chunk_body(t, kph)
            return 0
        lax.fori_loop(0, n_ch, chunk, 0)

        for tl in (n_ch - 1, n_ch - 2):
            for kph in range(4):
                @pl.when(tl % 4 == kph)
                def _(kph=kph):
                    wait_scatter(kph)

        plsc.subcore_barrier()

        def wchunk(kk, _):
            m = sid + kk * ns
            @pl.when(m < n_rch)
            def _():
                pltpu.sync_copy(acc.at[pl.ds(m * CHD, CHD)],
                                out_hbm.at[pl.ds(cid * n + m * CHD, CHD)])
            return 0
        lax.fori_loop(0, rch_per_sub, wchunk, 0)

    return k(dst).reshape(nc, n, h)


def kernel(x, edge_index, edge_attr, params):
    n, _ = x.shape
    e = edge_attr.shape[0]
    h = params["enc_w1"].shape[1]
    info = plsc.get_sparse_core_info()
    nc, ns = info.num_cores, info.num_subcores

    bn = 2000
    be = 4000

    src = edge_index[0]
    dst = edge_index[1]

    hv = _encoder(x, params["enc_w1"], params["enc_b1"],
                  params["enc_w2"], params["enc_b2"], bn)
    d2 = _sc_degree(dst, n, h, nc, ns)

    for lp in params["layers"]:
        wa = lp["mw1"][:h]
        wb = lp["mw1"][h:2 * h]
        wc = lp["mw1"][2 * h:]
        p, q = _pq(hv, wa, wb, bn)
        r = _r_term(edge_attr, params["ee_w"], params["ee_b"], wc,
                    lp["mb1"], be)
        s2 = _sc_segment_relu_sum(p, q, r, dst, src, nc, ns)
        hv = _node_update(hv, s2, d2, lp["mw2"], lp["mb2"],
                          lp["uw"][:h], lp["uw"][h:], lp["ub"],
                          lp["ln_g"], lp["ln_b"], bn)

    out = _decoder(hv, params["dec_w1"], params["dec_b1"],
                   params["dec_w2"], params["dec_b2"], bn)
    return out.reshape(n)
